# baseline (speedup 1.0000x reference)
import functools

import jax
import jax.numpy as jnp
from jax import lax
from jax.experimental import pallas as pl
from jax.experimental.pallas import tpu as pltpu
from jax.experimental.pallas import tpu_sc as plsc

N = 100000
NUM_EMBEDS = 118
D_IN = 128
DZ = 352
D_OUT = D_IN + DZ
CHUNK = 128
NW = 32
NCH_HI = 25
NCH_LO = 24
NW_HI = 782 - NW * NCH_LO
RPW_HI = NCH_HI * CHUNK
RPW_LO = NCH_LO * CHUNK
ZPAD = NW_HI * RPW_HI + (NW - NW_HI - 1) * RPW_LO + RPW_HI
TAIL = N - (N // CHUNK) * CHUNK
TAIL_BASE = (N // CHUNK) * CHUNK
BT = 1024


def _fuse_body(t_ref, w_ref, o_ref):
    scale = 1.0 / jnp.sqrt(jnp.float32(D_IN))
    o_ref[...] = jnp.dot(
        t_ref[...], w_ref[...], preferred_element_type=jnp.float32
    ) * scale


def _fused_table(table_p, w):
    return pl.pallas_call(
        _fuse_body,
        out_shape=jax.ShapeDtypeStruct((D_IN, D_IN), jnp.float32),
    )(table_p, w)


@functools.partial(
    pl.kernel,
    mesh=plsc.VectorSubcoreMesh(core_axis_name="c", subcore_axis_name="s"),
    out_type=jax.ShapeDtypeStruct((N, D_IN), jnp.float32),
    scratch_types=[
        pltpu.VMEM((RPW_HI,), jnp.int32),
        pltpu.VMEM((CHUNK, D_IN), jnp.float32),
        pltpu.VMEM((CHUNK, D_IN), jnp.float32),
        pltpu.VMEM((CHUNK, D_IN), jnp.float32),
        pltpu.VMEM((TAIL, D_IN), jnp.float32),
        pltpu.SemaphoreType.DMA,
        pltpu.SemaphoreType.DMA,
        pltpu.SemaphoreType.DMA,
        pltpu.SemaphoreType.DMA,
        pltpu.SemaphoreType.DMA,
        pltpu.SemaphoreType.DMA,
        pltpu.SemaphoreType.DMA,
    ],
)
def _sc_gather(z_hbm, fused_hbm, out_hbm,
               idx_v, r0, r1, r2, rt, g0, g1, g2, w0, w1, w2, ts):
    rows = [r0, r1, r2]
    gs = [g0, g1, g2]
    ws = [w0, w1, w2]

    wid = lax.axis_index("s") * 2 + lax.axis_index("c")
    is_hi = wid < NW_HI
    wbase = jnp.where(is_hi, wid * RPW_HI,
                      NW_HI * RPW_HI + (wid - NW_HI) * RPW_LO)
    pltpu.sync_copy(z_hbm.at[pl.ds(wbase, RPW_HI)], idx_v)

    bases = [wbase + j * CHUNK for j in range(NCH_HI)]
    full = [bases[j] + CHUNK <= N for j in range(NCH_HI)]
    tail = [jnp.logical_and(bases[j] <= TAIL_BASE,
                            TAIL_BASE < bases[j] + CHUNK)
            for j in range(NCH_HI)]
    valid_full = [full[j] if j < NCH_LO
                  else jnp.logical_and(is_hi, full[j])
                  for j in range(NCH_HI)]

    def fire_gather(j):
        p = j % 3
        pltpu.async_copy(
            fused_hbm.at[idx_v.at[pl.ds(j * CHUNK, CHUNK)]], rows[p], gs[p])

    def wait_gather(j):
        p = j % 3
        pltpu.make_async_copy(
            fused_hbm.at[idx_v.at[pl.ds(j * CHUNK, CHUNK)]],
            rows[p], gs[p]).wait()

    def fire_write(j):
        p = j % 3
        pltpu.async_copy(rows[p], out_hbm.at[pl.ds(bases[j], CHUNK)], ws[p])

    def wait_write(j):
        p = j % 3
        pltpu.make_async_copy(
            rows[p], out_hbm.at[pl.ds(bases[j], CHUNK)], ws[p]).wait()

    def guarded(pred, fn, *a):
        pl.when(pred)(lambda: fn(*a))

    guarded(valid_full[0], fire_gather, 0)
    for j in range(NCH_HI):
        nxt = j + 1
        if nxt < NCH_HI:
            if nxt >= 3:
                guarded(valid_full[nxt - 3], wait_write, nxt - 3)
            guarded(valid_full[nxt], fire_gather, nxt)

        def step(j=j):
            wait_gather(j)
            fire_write(j)

        guarded(valid_full[j], step)

        if j < NCH_LO:
            def tail_step(j=j):
                pltpu.async_copy(
                    fused_hbm.at[idx_v.at[pl.ds(j * CHUNK, TAIL)]], rt, ts)
                pltpu.make_async_copy(
                    fused_hbm.at[idx_v.at[pl.ds(j * CHUNK, TAIL)]],
                    rt, ts).wait()
                pltpu.sync_copy(rt, out_hbm.at[pl.ds(TAIL_BASE, TAIL)])

            guarded(tail[j], tail_step)
    for j in range(NCH_HI - 3, NCH_HI):
        guarded(valid_full[j], wait_write, j)


def _pad_body(x_ref, o_ref):
    t = jnp.transpose(x_ref[...], (1, 0))
    o_ref[...] = jnp.concatenate(
        [t, jnp.zeros((DZ, BT), jnp.float32)], axis=0)


def _transpose_pad(out0e):
    nblk = -(-N // BT)
    return pl.pallas_call(
        _pad_body,
        grid=(nblk,),
        in_specs=[pl.BlockSpec((BT, D_IN), lambda i: (i, 0))],
        out_specs=pl.BlockSpec((D_OUT, BT), lambda i: (0, i)),
        out_shape=jax.ShapeDtypeStruct((D_OUT, N), jnp.float32),
    )(out0e)


def kernel(z, table, W):
    z32 = z.astype(jnp.int32)
    z_pad = jnp.pad(z32, (0, ZPAD - N))
    table_p = jnp.zeros((D_IN, D_IN), jnp.float32).at[:NUM_EMBEDS].set(table)
    fused = _fused_table(table_p, W)
    out0e = _sc_gather(z_pad, fused)
    return _transpose_pad(out0e).T


# speedup vs baseline: 1.0045x; 1.0045x over previous
import functools

import jax
import jax.numpy as jnp
from jax import lax
from jax.experimental import pallas as pl
from jax.experimental.pallas import tpu as pltpu
from jax.experimental.pallas import tpu_sc as plsc

N = 100000
NUM_EMBEDS = 118
D_IN = 128
DZ = 352
D_OUT = D_IN + DZ
CHUNK = 256
HALF = 128
NW = 32
NCH_HI = 13
NCH_LO = 12
NW_HI = 390 - NW * NCH_LO
RPW_HI = NCH_HI * CHUNK
RPW_LO = NCH_LO * CHUNK
ZPAD = NW_HI * RPW_HI + (NW - NW_HI - 1) * RPW_LO + RPW_HI
TAIL = N - (N // CHUNK) * CHUNK
TAIL_BASE = (N // CHUNK) * CHUNK
BT = 1024


def _fuse_body(t_ref, w_ref, o_ref):
    scale = 1.0 / jnp.sqrt(jnp.float32(D_IN))
    o_ref[...] = jnp.dot(
        t_ref[...], w_ref[...], preferred_element_type=jnp.float32
    ) * scale


def _fused_table(table_p, w):
    return pl.pallas_call(
        _fuse_body,
        out_shape=jax.ShapeDtypeStruct((D_IN, D_IN), jnp.float32),
    )(table_p, w)


@functools.partial(
    pl.kernel,
    mesh=plsc.VectorSubcoreMesh(core_axis_name="c", subcore_axis_name="s"),
    out_type=jax.ShapeDtypeStruct((N, D_IN), jnp.float32),
    scratch_types=[
        pltpu.VMEM((RPW_HI,), jnp.int32),
        pltpu.VMEM((CHUNK, D_IN), jnp.float32),
        pltpu.VMEM((CHUNK, D_IN), jnp.float32),
        pltpu.VMEM((TAIL, D_IN), jnp.float32),
        pltpu.SemaphoreType.DMA,
        pltpu.SemaphoreType.DMA,
        pltpu.SemaphoreType.DMA,
        pltpu.SemaphoreType.DMA,
        pltpu.SemaphoreType.DMA,
    ],
)
def _sc_gather(z_hbm, fused_hbm, out_hbm,
               idx_v, r0, r1, rt, g0, g1, w0, w1, ts):
    rows = [r0, r1]
    gs = [g0, g1]
    ws = [w0, w1]

    wid = lax.axis_index("s") * 2 + lax.axis_index("c")
    is_hi = wid < NW_HI
    wbase = jnp.where(is_hi, wid * RPW_HI,
                      NW_HI * RPW_HI + (wid - NW_HI) * RPW_LO)
    pltpu.sync_copy(z_hbm.at[pl.ds(wbase, RPW_HI)], idx_v)

    bases = [wbase + j * CHUNK for j in range(NCH_HI)]
    full = [bases[j] + CHUNK <= N for j in range(NCH_HI)]
    valid_full = [full[j] if j < NCH_LO
                  else jnp.logical_and(is_hi, full[j])
                  for j in range(NCH_HI)]

    def fire_gather(j):
        p = j % 2
        pltpu.async_copy(
            fused_hbm.at[idx_v.at[pl.ds(j * CHUNK, HALF)]],
            rows[p].at[pl.ds(0, HALF)], gs[p])
        pltpu.async_copy(
            fused_hbm.at[idx_v.at[pl.ds(j * CHUNK + HALF, HALF)]],
            rows[p].at[pl.ds(HALF, HALF)], gs[p])

    def wait_gather(j):
        p = j % 2
        pltpu.make_async_copy(
            fused_hbm.at[idx_v.at[pl.ds(j * CHUNK, HALF)]],
            rows[p].at[pl.ds(0, HALF)], gs[p]).wait()
        pltpu.make_async_copy(
            fused_hbm.at[idx_v.at[pl.ds(j * CHUNK + HALF, HALF)]],
            rows[p].at[pl.ds(HALF, HALF)], gs[p]).wait()

    def fire_write(j):
        p = j % 2
        pltpu.async_copy(rows[p], out_hbm.at[pl.ds(bases[j], CHUNK)], ws[p])

    def wait_write(j):
        p = j % 2
        pltpu.make_async_copy(
            rows[p], out_hbm.at[pl.ds(bases[j], CHUNK)], ws[p]).wait()

    def guarded(pred, fn, *a):
        pl.when(pred)(lambda: fn(*a))

    guarded(valid_full[0], fire_gather, 0)
    for j in range(NCH_HI):
        nxt = j + 1
        if nxt < NCH_HI:
            if nxt >= 2:
                guarded(valid_full[nxt - 2], wait_write, nxt - 2)
            guarded(valid_full[nxt], fire_gather, nxt)

        def step(j=j):
            wait_gather(j)
            fire_write(j)

        guarded(valid_full[j], step)

    def tail_step():
        pltpu.async_copy(
            fused_hbm.at[idx_v.at[pl.ds(NCH_LO * CHUNK, HALF)]],
            rt.at[pl.ds(0, HALF)], ts)
        pltpu.async_copy(
            fused_hbm.at[idx_v.at[pl.ds(NCH_LO * CHUNK + HALF, TAIL - HALF)]],
            rt.at[pl.ds(HALF, TAIL - HALF)], ts)
        pltpu.make_async_copy(
            fused_hbm.at[idx_v.at[pl.ds(NCH_LO * CHUNK, HALF)]],
            rt.at[pl.ds(0, HALF)], ts).wait()
        pltpu.make_async_copy(
            fused_hbm.at[idx_v.at[pl.ds(NCH_LO * CHUNK + HALF, TAIL - HALF)]],
            rt.at[pl.ds(HALF, TAIL - HALF)], ts).wait()
        pltpu.sync_copy(rt, out_hbm.at[pl.ds(TAIL_BASE, TAIL)])

    guarded(wid == NW - 1, tail_step)
    for j in range(NCH_HI - 2, NCH_HI):
        guarded(valid_full[j], wait_write, j)


def _pad_body(x_ref, o_ref):
    t = jnp.transpose(x_ref[...], (1, 0))
    o_ref[...] = jnp.concatenate(
        [t, jnp.zeros((DZ, BT), jnp.float32)], axis=0)


def _transpose_pad(out0e):
    nblk = -(-N // BT)
    return pl.pallas_call(
        _pad_body,
        grid=(nblk,),
        in_specs=[pl.BlockSpec((BT, D_IN), lambda i: (i, 0))],
        out_specs=pl.BlockSpec((D_OUT, BT), lambda i: (0, i)),
        out_shape=jax.ShapeDtypeStruct((D_OUT, N), jnp.float32),
    )(out0e)


def kernel(z, table, W):
    z32 = z.astype(jnp.int32)
    z_pad = jnp.pad(z32, (0, ZPAD - N))
    table_p = jnp.zeros((D_IN, D_IN), jnp.float32).at[:NUM_EMBEDS].set(table)
    fused = _fused_table(table_p, W)
    out0e = _sc_gather(z_pad, fused)
    return _transpose_pad(out0e).T


# speedup vs baseline: 1.4536x; 1.4471x over previous
import functools

import jax
import jax.numpy as jnp
from jax import lax
from jax.experimental import pallas as pl
from jax.experimental.pallas import tpu as pltpu
from jax.experimental.pallas import tpu_sc as plsc

N = 100000
NUM_EMBEDS = 118
D_IN = 128
DZ = 352
D_OUT = D_IN + DZ
CHUNK = 256
HALF = 128
NW = 32
NCH_HI = 13
NCH_LO = 12
NW_HI = 390 - NW * NCH_LO
RPW_HI = NCH_HI * CHUNK
RPW_LO = NCH_LO * CHUNK
ZPAD = NW_HI * RPW_HI + (NW - NW_HI - 1) * RPW_LO + RPW_HI
TAIL = N - (N // CHUNK) * CHUNK
TAIL_BASE = (N // CHUNK) * CHUNK
BT = 1024


def _fuse_body(t_ref, w_ref, o_ref):
    scale = 1.0 / jnp.sqrt(jnp.float32(D_IN))
    o_ref[...] = jnp.dot(
        t_ref[...], w_ref[...], preferred_element_type=jnp.float32
    ) * scale


def _fused_table(table_p, w):
    return pl.pallas_call(
        _fuse_body,
        out_shape=jax.ShapeDtypeStruct((D_IN, D_IN), jnp.float32),
    )(table_p, w)


@functools.partial(
    pl.kernel,
    mesh=plsc.VectorSubcoreMesh(core_axis_name="c", subcore_axis_name="s"),
    out_type=jax.ShapeDtypeStruct((N, D_IN), jnp.float32),
    scratch_types=[
        pltpu.VMEM((RPW_HI,), jnp.int32),
        pltpu.VMEM_SHARED((D_IN, D_IN), jnp.float32),
        pltpu.VMEM((CHUNK, D_IN), jnp.float32),
        pltpu.VMEM((CHUNK, D_IN), jnp.float32),
        pltpu.VMEM((TAIL, D_IN), jnp.float32),
        pltpu.SemaphoreType.DMA,
        pltpu.SemaphoreType.DMA,
        pltpu.SemaphoreType.DMA,
        pltpu.SemaphoreType.DMA,
        pltpu.SemaphoreType.DMA,
    ],
)
def _sc_gather(z_hbm, fused_hbm, out_hbm,
               idx_v, fused_v, r0, r1, rt, g0, g1, w0, w1, ts):
    rows = [r0, r1]
    gs = [g0, g1]
    ws = [w0, w1]

    wid = lax.axis_index("s") * 2 + lax.axis_index("c")
    is_hi = wid < NW_HI
    wbase = jnp.where(is_hi, wid * RPW_HI,
                      NW_HI * RPW_HI + (wid - NW_HI) * RPW_LO)
    pltpu.sync_copy(z_hbm.at[pl.ds(wbase, RPW_HI)], idx_v)
    pltpu.sync_copy(fused_hbm, fused_v)

    bases = [wbase + j * CHUNK for j in range(NCH_HI)]
    full = [bases[j] + CHUNK <= N for j in range(NCH_HI)]
    valid_full = [full[j] if j < NCH_LO
                  else jnp.logical_and(is_hi, full[j])
                  for j in range(NCH_HI)]

    def fire_gather(j):
        p = j % 2
        pltpu.async_copy(
            fused_v.at[idx_v.at[pl.ds(j * CHUNK, HALF)]],
            rows[p].at[pl.ds(0, HALF)], gs[p])
        pltpu.async_copy(
            fused_v.at[idx_v.at[pl.ds(j * CHUNK + HALF, HALF)]],
            rows[p].at[pl.ds(HALF, HALF)], gs[p])

    def wait_gather(j):
        p = j % 2
        pltpu.make_async_copy(
            fused_v.at[idx_v.at[pl.ds(j * CHUNK, HALF)]],
            rows[p].at[pl.ds(0, HALF)], gs[p]).wait()
        pltpu.make_async_copy(
            fused_v.at[idx_v.at[pl.ds(j * CHUNK + HALF, HALF)]],
            rows[p].at[pl.ds(HALF, HALF)], gs[p]).wait()

    def fire_write(j):
        p = j % 2
        pltpu.async_copy(rows[p], out_hbm.at[pl.ds(bases[j], CHUNK)], ws[p])

    def wait_write(j):
        p = j % 2
        pltpu.make_async_copy(
            rows[p], out_hbm.at[pl.ds(bases[j], CHUNK)], ws[p]).wait()

    def guarded(pred, fn, *a):
        pl.when(pred)(lambda: fn(*a))

    guarded(valid_full[0], fire_gather, 0)
    for j in range(NCH_HI):
        nxt = j + 1
        if nxt < NCH_HI:
            if nxt >= 2:
                guarded(valid_full[nxt - 2], wait_write, nxt - 2)
            guarded(valid_full[nxt], fire_gather, nxt)

        def step(j=j):
            wait_gather(j)
            fire_write(j)

        guarded(valid_full[j], step)

    def tail_step():
        pltpu.async_copy(
            fused_v.at[idx_v.at[pl.ds(NCH_LO * CHUNK, HALF)]],
            rt.at[pl.ds(0, HALF)], ts)
        pltpu.async_copy(
            fused_v.at[idx_v.at[pl.ds(NCH_LO * CHUNK + HALF, TAIL - HALF)]],
            rt.at[pl.ds(HALF, TAIL - HALF)], ts)
        pltpu.make_async_copy(
            fused_v.at[idx_v.at[pl.ds(NCH_LO * CHUNK, HALF)]],
            rt.at[pl.ds(0, HALF)], ts).wait()
        pltpu.make_async_copy(
            fused_v.at[idx_v.at[pl.ds(NCH_LO * CHUNK + HALF, TAIL - HALF)]],
            rt.at[pl.ds(HALF, TAIL - HALF)], ts).wait()
        pltpu.sync_copy(rt, out_hbm.at[pl.ds(TAIL_BASE, TAIL)])

    guarded(wid == NW - 1, tail_step)
    for j in range(NCH_HI - 2, NCH_HI):
        guarded(valid_full[j], wait_write, j)


def _pad_body(x_ref, o_ref):
    t = jnp.transpose(x_ref[...], (1, 0))
    o_ref[...] = jnp.concatenate(
        [t, jnp.zeros((DZ, BT), jnp.float32)], axis=0)


def _transpose_pad(out0e):
    nblk = -(-N // BT)
    return pl.pallas_call(
        _pad_body,
        grid=(nblk,),
        in_specs=[pl.BlockSpec((BT, D_IN), lambda i: (i, 0))],
        out_specs=pl.BlockSpec((D_OUT, BT), lambda i: (0, i)),
        out_shape=jax.ShapeDtypeStruct((D_OUT, N), jnp.float32),
    )(out0e)


def kernel(z, table, W):
    z32 = z.astype(jnp.int32)
    z_pad = jnp.pad(z32, (0, ZPAD - N))
    table_p = jnp.zeros((D_IN, D_IN), jnp.float32).at[:NUM_EMBEDS].set(table)
    fused = _fused_table(table_p, W)
    out0e = _sc_gather(z_pad, fused)
    return _transpose_pad(out0e).T
